# baseline (speedup 1.0000x reference)
import functools

import jax
import jax.numpy as jnp
from jax import lax
from jax.experimental import pallas as pl
from jax.experimental.pallas import tpu as pltpu
from jax.experimental.pallas import tpu_sc as plsc

N = 50000
D = 256
MAX_DEG = 10
DEG0 = 5000
DEGS = 4500

SC_NC = 2
SC_NS = 16
SC_NW = SC_NC * SC_NS

SC_CH = {1: 100, 2: 60, 3: 36, 4: 30, 5: 25, 6: 20, 7: 18, 8: 15, 9: 12, 10: 12}

R = 500
NBLK = N // R
SEG0_BLKS = DEG0 // R
SEGD_BLKS = DEGS // R


def _seg(i):
    return jnp.where(i < SEG0_BLKS, 0, 1 + (i - SEG0_BLKS) // SEGD_BLKS)


def _tc_body(s_ref, x_ref, wr_ref, ws_ref, bc_ref, o_ref):
    acc = jnp.dot(s_ref[0], wr_ref[0], preferred_element_type=jnp.float32)
    acc = acc + jnp.dot(x_ref[0], ws_ref[0], preferred_element_type=jnp.float32)
    o_ref[0] = jnp.maximum(acc + bc_ref[0], 0.0)


def _tc_call(S, X, Wr, Ws, bc):
    out = pl.pallas_call(
        _tc_body,
        grid=(NBLK,),
        in_specs=[
            pl.BlockSpec((1, R, D), lambda i: (jnp.maximum(i - SEG0_BLKS, 0), 0, 0)),
            pl.BlockSpec((1, R, D), lambda i: (i, 0, 0)),
            pl.BlockSpec((1, D, D), lambda i: (_seg(i), 0, 0)),
            pl.BlockSpec((1, D, D), lambda i: (_seg(i), 0, 0)),
            pl.BlockSpec((1, 1, D), lambda i: (_seg(i), 0, 0)),
        ],
        out_specs=pl.BlockSpec((1, R, D), lambda i: (i, 0, 0)),
        out_shape=jax.ShapeDtypeStruct((NBLK, R, D), jnp.float32),
    )(S.reshape(-1, R, D), X.reshape(NBLK, R, D), Wr, Ws, bc.reshape(-1, 1, D))
    return out.reshape(N, D)


def _sc_geom(d):
    C = SC_CH[d]
    n = DEGS // C
    cpt = -(-n // SC_NW)
    cpt += cpt % 2
    return C, n, cpt


def _sc_body(feat, *rest):
    adjrs = rest[:MAX_DEG]
    s_out = rest[MAX_DEG]
    ix0, ix1, gbA, gbB, obuf, semA, semB, semI0, semI1 = rest[MAX_DEG + 1:]
    ixs = (ix0, ix1)
    semIs = (semI0, semI1)
    wid = lax.axis_index("s") * SC_NC + lax.axis_index("c")

    def idx_issue(d):
        C, n, cpt = _sc_geom(d)
        start = jnp.minimum(wid * cpt, n - cpt)
        par = (d - 1) % 2
        src = adjrs[d - 1].at[pl.ds(start, cpt)]
        dst = ixs[par].at[pl.ds(0, cpt)]
        return pltpu.async_copy(src, dst, semIs[par]), start

    cp, start1 = idx_issue(1)
    starts = {1: start1}
    pending = {1: cp}
    for d in range(1, MAX_DEG + 1):
        C, n, cpt = _sc_geom(d)
        par = (d - 1) % 2
        ix = ixs[par]
        start = starts[d]
        pending.pop(d).wait()
        if d < MAX_DEG:
            cp, s2 = idx_issue(d + 1)
            starts[d + 1] = s2
            pending[d + 1] = cp
        npairs = cpt // 2

        def gather(t, gb, sem, ix=ix):
            pltpu.async_copy(feat.at[ix.at[t]], gb, sem)

        def gwait(t, gb, sem, ix=ix):
            pltpu.make_async_copy(feat.at[ix.at[t]], gb, sem).wait()

        def sumstore(t, gb, d=d, C=C, start=start):
            orow = (d - 1) * DEGS + (start + t) * C
            if d == 1:
                pltpu.sync_copy(gb.at[pl.ds(0, C)], s_out.at[pl.ds(orow, C)])
            else:
                def body_j(j, _, d=d, gb=gb):
                    rb = j * d
                    for k in range(D // 16):
                        sl = pl.ds(k * 16, 16)
                        acc = gb[rb, sl]
                        for t2 in range(1, d):
                            acc = acc + gb[rb + t2, sl]
                        obuf[j, sl] = acc
                    return 0
                lax.fori_loop(0, C, body_j, 0)
                pltpu.sync_copy(obuf.at[pl.ds(0, C)], s_out.at[pl.ds(orow, C)])

        gather(0, gbA, semA)

        def pair(p, _, d=d, C=C, npairs=npairs):
            t0 = 2 * p
            gather(t0 + 1, gbB, semB)
            gwait(t0, gbA, semA)
            sumstore(t0, gbA)

            @pl.when(p < npairs - 1)
            def _():
                gather(t0 + 2, gbA, semA)

            gwait(t0 + 1, gbB, semB)
            sumstore(t0 + 1, gbB)
            return 0

        lax.fori_loop(0, npairs, pair, 0)


def _sc_call(feat, adjs):
    adjrs = [
        jnp.pad(a.reshape(DEGS // SC_CH[d + 1], SC_CH[d + 1] * (d + 1)),
                ((0, 0), (0, 128 - SC_CH[d + 1] * (d + 1))))
        for d, a in enumerate(adjs)
    ]
    maxcpt = max(_sc_geom(d)[2] for d in range(1, MAX_DEG + 1))
    fn = pl.kernel(
        _sc_body,
        out_type=jax.ShapeDtypeStruct((MAX_DEG * DEGS, D), jnp.float32),
        mesh=plsc.VectorSubcoreMesh(core_axis_name="c", subcore_axis_name="s"),
        scratch_types=[
            pltpu.VMEM((maxcpt, 128), jnp.int32),
            pltpu.VMEM((maxcpt, 128), jnp.int32),
            pltpu.VMEM((128, D), jnp.float32),
            pltpu.VMEM((128, D), jnp.float32),
            pltpu.VMEM((60, D), jnp.float32),
            pltpu.SemaphoreType.DMA,
            pltpu.SemaphoreType.DMA,
            pltpu.SemaphoreType.DMA,
            pltpu.SemaphoreType.DMA,
        ],
        compiler_params=pltpu.CompilerParams(use_tc_tiling_on_sc=False),
    )
    return fn(feat, *adjrs)


def kernel(atom_features, deg_slice, adj_1, adj_2, adj_3, adj_4, adj_5,
           adj_6, adj_7, adj_8, adj_9, adj_10, W, b):
    adjs = [adj_1, adj_2, adj_3, adj_4, adj_5, adj_6, adj_7, adj_8, adj_9, adj_10]
    S = _sc_call(atom_features, adjs)

    Wr = jnp.concatenate([jnp.zeros((1, D, D), jnp.float32), W[0:20:2]], axis=0)
    Ws = jnp.concatenate([W[20:21], W[1:20:2]], axis=0)
    bc = jnp.concatenate([b[20:21], b[0:20:2] + b[1:20:2]], axis=0)
    return _tc_call(S, atom_features, Wr, Ws, bc)


# speedup vs baseline: 1.0013x; 1.0013x over previous
import functools

import jax
import jax.numpy as jnp
from jax import lax
from jax.experimental import pallas as pl
from jax.experimental.pallas import tpu as pltpu
from jax.experimental.pallas import tpu_sc as plsc

N = 50000
D = 256
MAX_DEG = 10
DEG0 = 5000
DEGS = 4500

SC_NC = 2
SC_NS = 16
SC_NW = SC_NC * SC_NS

SC_CH = {1: 100, 2: 60, 3: 36, 4: 30, 5: 25, 6: 20, 7: 18, 8: 15, 9: 12, 10: 12}

R = 500
NBLK = N // R
SEG0_BLKS = DEG0 // R
SEGD_BLKS = DEGS // R


def _seg(i):
    return jnp.where(i < SEG0_BLKS, 0, 1 + (i - SEG0_BLKS) // SEGD_BLKS)


def _tc_body(s_ref, x_ref, wr_ref, ws_ref, bc_ref, o_ref):
    acc = jnp.dot(s_ref[0], wr_ref[0], preferred_element_type=jnp.float32)
    acc = acc + jnp.dot(x_ref[0], ws_ref[0], preferred_element_type=jnp.float32)
    o_ref[0] = jnp.maximum(acc + bc_ref[0], 0.0)


def _tc_call(S, X, Wr, Ws, bc):
    out = pl.pallas_call(
        _tc_body,
        grid=(NBLK,),
        in_specs=[
            pl.BlockSpec((1, R, D), lambda i: (jnp.maximum(i - SEG0_BLKS, 0), 0, 0)),
            pl.BlockSpec((1, R, D), lambda i: (i, 0, 0)),
            pl.BlockSpec((1, D, D), lambda i: (_seg(i), 0, 0)),
            pl.BlockSpec((1, D, D), lambda i: (_seg(i), 0, 0)),
            pl.BlockSpec((1, 1, D), lambda i: (_seg(i), 0, 0)),
        ],
        out_specs=pl.BlockSpec((1, R, D), lambda i: (i, 0, 0)),
        out_shape=jax.ShapeDtypeStruct((NBLK, R, D), jnp.float32),
    )(S.reshape(-1, R, D), X.reshape(NBLK, R, D), Wr, Ws, bc.reshape(-1, 1, D))
    return out.reshape(N, D)


def _sc_geom(d):
    C = SC_CH[d]
    n = DEGS // C
    cpt = -(-n // SC_NW)
    cpt += cpt % 2
    return C, n, cpt


def _sc_body(feat, *rest):
    adjrs = rest[:MAX_DEG]
    s_out = rest[MAX_DEG]
    ix0, ix1, gbA, gbB, obuf, semA, semB, semI0, semI1 = rest[MAX_DEG + 1:]
    ixs = (ix0, ix1)
    semIs = (semI0, semI1)
    wid = lax.axis_index("s") * SC_NC + lax.axis_index("c")

    def idx_issue(d):
        C, n, cpt = _sc_geom(d)
        start = jnp.minimum(wid * cpt, n - cpt)
        par = (d - 1) % 2
        src = adjrs[d - 1].at[pl.ds(start, cpt)]
        dst = ixs[par].at[pl.ds(0, cpt)]
        return pltpu.async_copy(src, dst, semIs[par]), start

    cp, start1 = idx_issue(1)
    starts = {1: start1}
    pending = {1: cp}
    for d in range(1, MAX_DEG + 1):
        C, n, cpt = _sc_geom(d)
        par = (d - 1) % 2
        ix = ixs[par]
        start = starts[d]
        pending.pop(d).wait()
        if d < MAX_DEG:
            cp, s2 = idx_issue(d + 1)
            starts[d + 1] = s2
            pending[d + 1] = cp
        npairs = cpt // 2

        def gather(t, gb, sem, ix=ix):
            pltpu.async_copy(feat.at[ix.at[t]], gb, sem)

        def gwait(t, gb, sem, ix=ix):
            pltpu.make_async_copy(feat.at[ix.at[t]], gb, sem).wait()

        def sumstore(t, gb, d=d, C=C, start=start):
            orow = (d - 1) * DEGS + (start + t) * C
            if d == 1:
                pltpu.sync_copy(gb.at[pl.ds(0, C)], s_out.at[pl.ds(orow, C)])
            else:
                def body_j(j, _, d=d, gb=gb):
                    rb = j * d
                    for k in range(D // 16):
                        sl = pl.ds(k * 16, 16)
                        acc = gb[rb, sl]
                        for t2 in range(1, d):
                            acc = acc + gb[rb + t2, sl]
                        obuf[j, sl] = acc
                    return 0
                lax.fori_loop(0, C, body_j, 0)
                pltpu.sync_copy(obuf.at[pl.ds(0, C)], s_out.at[pl.ds(orow, C)])

        def chunk_one(t, _):
            gather(t, gbA, semA)
            gwait(t, gbA, semA)
            sumstore(t, gbA)
            return 0

        lax.fori_loop(0, 2 * npairs, chunk_one, 0)


def _sc_call(feat, adjs):
    adjrs = [
        jnp.pad(a.reshape(DEGS // SC_CH[d + 1], SC_CH[d + 1] * (d + 1)),
                ((0, 0), (0, 128 - SC_CH[d + 1] * (d + 1))))
        for d, a in enumerate(adjs)
    ]
    maxcpt = max(_sc_geom(d)[2] for d in range(1, MAX_DEG + 1))
    fn = pl.kernel(
        _sc_body,
        out_type=jax.ShapeDtypeStruct((MAX_DEG * DEGS, D), jnp.float32),
        mesh=plsc.VectorSubcoreMesh(core_axis_name="c", subcore_axis_name="s"),
        scratch_types=[
            pltpu.VMEM((maxcpt, 128), jnp.int32),
            pltpu.VMEM((maxcpt, 128), jnp.int32),
            pltpu.VMEM((128, D), jnp.float32),
            pltpu.VMEM((128, D), jnp.float32),
            pltpu.VMEM((60, D), jnp.float32),
            pltpu.SemaphoreType.DMA,
            pltpu.SemaphoreType.DMA,
            pltpu.SemaphoreType.DMA,
            pltpu.SemaphoreType.DMA,
        ],
        compiler_params=pltpu.CompilerParams(use_tc_tiling_on_sc=False),
    )
    return fn(feat, *adjrs)


def kernel(atom_features, deg_slice, adj_1, adj_2, adj_3, adj_4, adj_5,
           adj_6, adj_7, adj_8, adj_9, adj_10, W, b):
    adjs = [adj_1, adj_2, adj_3, adj_4, adj_5, adj_6, adj_7, adj_8, adj_9, adj_10]
    S = _sc_call(atom_features, adjs)

    Wr = jnp.concatenate([jnp.zeros((1, D, D), jnp.float32), W[0:20:2]], axis=0)
    Ws = jnp.concatenate([W[20:21], W[1:20:2]], axis=0)
    bc = jnp.concatenate([b[20:21], b[0:20:2] + b[1:20:2]], axis=0)
    return _tc_call(S, atom_features, Wr, Ws, bc)


# speedup vs baseline: 2.5966x; 2.5934x over previous
import functools

import jax
import jax.numpy as jnp
from jax import lax
from jax.experimental import pallas as pl
from jax.experimental.pallas import tpu as pltpu
from jax.experimental.pallas import tpu_sc as plsc

N = 50000
D = 256
MAX_DEG = 10
DEG0 = 5000
DEGS = 4500

SC_NC = 2
SC_NS = 16
SC_NW = SC_NC * SC_NS

SC_CH = {1: 100, 2: 60, 3: 36, 4: 30, 5: 25, 6: 20, 7: 18, 8: 15, 9: 12, 10: 12}

R = 500
NBLK = N // R
SEG0_BLKS = DEG0 // R
SEGD_BLKS = DEGS // R


def _seg(i):
    return jnp.where(i < SEG0_BLKS, 0, 1 + (i - SEG0_BLKS) // SEGD_BLKS)


def _tc_body(s_ref, x_ref, wr_ref, ws_ref, bc_ref, o_ref):
    acc = jnp.dot(s_ref[0], wr_ref[0], preferred_element_type=jnp.float32)
    acc = acc + jnp.dot(x_ref[0], ws_ref[0], preferred_element_type=jnp.float32)
    o_ref[0] = jnp.maximum(acc + bc_ref[0], 0.0)


def _tc_call(S, X, Wr, Ws, bc):
    out = pl.pallas_call(
        _tc_body,
        grid=(NBLK,),
        in_specs=[
            pl.BlockSpec((1, R, D), lambda i: (jnp.maximum(i - SEG0_BLKS, 0), 0, 0)),
            pl.BlockSpec((1, R, D), lambda i: (i, 0, 0)),
            pl.BlockSpec((1, D, D), lambda i: (_seg(i), 0, 0)),
            pl.BlockSpec((1, D, D), lambda i: (_seg(i), 0, 0)),
            pl.BlockSpec((1, 1, D), lambda i: (_seg(i), 0, 0)),
        ],
        out_specs=pl.BlockSpec((1, R, D), lambda i: (i, 0, 0)),
        out_shape=jax.ShapeDtypeStruct((NBLK, R, D), jnp.float32),
    )(S.reshape(-1, R, D), X.reshape(NBLK, R, D), Wr, Ws, bc.reshape(-1, 1, D))
    return out.reshape(N, D)


def _sc_geom(d):
    C = SC_CH[d]
    n = DEGS // C
    cpt = -(-n // SC_NW)
    cpt += cpt % 2
    return C, n, cpt


def _sc_body(feat, *rest):
    adjrs = rest[:MAX_DEG]
    s_out = rest[MAX_DEG]
    scratch = rest[MAX_DEG + 1:]
    ixs = scratch[:MAX_DEG]
    gbA, gbB, obuf, semI, semA, semB = scratch[MAX_DEG:]
    wid = lax.axis_index("s") * SC_NC + lax.axis_index("c")

    starts = {}
    for d in range(1, MAX_DEG + 1):
        C, n, cpt = _sc_geom(d)
        start = jnp.minimum(wid * cpt, n - cpt)
        starts[d] = start
        pltpu.async_copy(adjrs[d - 1].at[pl.ds(start, cpt)], ixs[d - 1], semI)
    for d in range(1, MAX_DEG + 1):
        C, n, cpt = _sc_geom(d)
        pltpu.make_async_copy(
            adjrs[d - 1].at[pl.ds(starts[d], cpt)], ixs[d - 1], semI).wait()

    for d in range(1, MAX_DEG + 1):
        C, n, cpt = _sc_geom(d)
        ix = ixs[d - 1]
        start = starts[d]
        npairs = cpt // 2

        def gather(t, gb, sem, d=d, C=C, ix=ix):
            pltpu.async_copy(feat.at[ix.at[t]], gb.at[pl.ds(0, C * d)], sem)

        def gwait(t, gb, sem, d=d, C=C, ix=ix):
            pltpu.make_async_copy(
                feat.at[ix.at[t]], gb.at[pl.ds(0, C * d)], sem).wait()

        def sumstore(t, gb, d=d, C=C, start=start):
            orow = (d - 1) * DEGS + (start + t) * C
            if d == 1:
                pltpu.sync_copy(gb.at[pl.ds(0, C)], s_out.at[pl.ds(orow, C)])
            else:
                def body_j(j, _, d=d, gb=gb):
                    rb = j * d
                    for k in range(D // 16):
                        sl = pl.ds(k * 16, 16)
                        acc = gb[rb, sl]
                        for t2 in range(1, d):
                            acc = acc + gb[rb + t2, sl]
                        obuf[j, sl] = acc
                    return 0
                lax.fori_loop(0, C, body_j, 0)
                pltpu.sync_copy(obuf.at[pl.ds(0, C)], s_out.at[pl.ds(orow, C)])

        gather(0, gbA, semA)

        def pair(p, _, npairs=npairs):
            t0 = 2 * p
            gather(t0 + 1, gbB, semB)
            gwait(t0, gbA, semA)
            sumstore(t0, gbA)

            @pl.when(p < npairs - 1)
            def _():
                gather(t0 + 2, gbA, semA)

            gwait(t0 + 1, gbB, semB)
            sumstore(t0 + 1, gbB)
            return 0

        lax.fori_loop(0, npairs, pair, 0)


def _sc_call(feat, adjs):
    adjrs = [a.reshape(DEGS // SC_CH[d + 1], SC_CH[d + 1] * (d + 1))
             for d, a in enumerate(adjs)]
    ix_types = []
    for d in range(1, MAX_DEG + 1):
        C, _, cpt = _sc_geom(d)
        ix_types.append(pltpu.VMEM((cpt, C * d), jnp.int32))
    fn = pl.kernel(
        _sc_body,
        out_type=jax.ShapeDtypeStruct((MAX_DEG * DEGS, D), jnp.float32),
        mesh=plsc.VectorSubcoreMesh(core_axis_name="c", subcore_axis_name="s"),
        scratch_types=ix_types + [
            pltpu.VMEM((128, D), jnp.float32),
            pltpu.VMEM((128, D), jnp.float32),
            pltpu.VMEM((60, D), jnp.float32),
            pltpu.SemaphoreType.DMA,
            pltpu.SemaphoreType.DMA,
            pltpu.SemaphoreType.DMA,
        ],
        compiler_params=pltpu.CompilerParams(use_tc_tiling_on_sc=False),
    )
    return fn(feat, *adjrs)


def kernel(atom_features, deg_slice, adj_1, adj_2, adj_3, adj_4, adj_5,
           adj_6, adj_7, adj_8, adj_9, adj_10, W, b):
    adjs = [adj_1, adj_2, adj_3, adj_4, adj_5, adj_6, adj_7, adj_8, adj_9, adj_10]
    S = _sc_call(atom_features, adjs)

    Wr = jnp.concatenate([jnp.zeros((1, D, D), jnp.float32), W[0:20:2]], axis=0)
    Ws = jnp.concatenate([W[20:21], W[1:20:2]], axis=0)
    bc = jnp.concatenate([b[20:21], b[0:20:2] + b[1:20:2]], axis=0)
    return _tc_call(S, atom_features, Wr, Ws, bc)


# speedup vs baseline: 2.6816x; 1.0327x over previous
import functools

import jax
import jax.numpy as jnp
from jax import lax
from jax.experimental import pallas as pl
from jax.experimental.pallas import tpu as pltpu
from jax.experimental.pallas import tpu_sc as plsc

N = 50000
D = 256
MAX_DEG = 10
DEG0 = 5000
DEGS = 4500

SC_NC = 2
SC_NS = 16
SC_NW = SC_NC * SC_NS

SC_CH = {1: 60, 2: 60, 3: 36, 4: 30, 5: 25, 6: 20, 7: 18, 8: 15, 9: 12, 10: 12}

R = 500
NBLK = N // R
SEG0_BLKS = DEG0 // R
SEGD_BLKS = DEGS // R


def _seg(i):
    return jnp.where(i < SEG0_BLKS, 0, 1 + (i - SEG0_BLKS) // SEGD_BLKS)


def _tc_self_body(x_ref, ws_ref, bc_ref, p_ref):
    acc = jnp.dot(x_ref[0], ws_ref[0], preferred_element_type=jnp.float32)
    p_ref[0] = acc + bc_ref[0]


def _tc_self_call(X, Ws, bc):
    return pl.pallas_call(
        _tc_self_body,
        grid=(NBLK,),
        in_specs=[
            pl.BlockSpec((1, R, D), lambda i: (i, 0, 0)),
            pl.BlockSpec((1, D, D), lambda i: (_seg(i), 0, 0)),
            pl.BlockSpec((1, 1, D), lambda i: (_seg(i), 0, 0)),
        ],
        out_specs=pl.BlockSpec((1, R, D), lambda i: (i, 0, 0)),
        out_shape=jax.ShapeDtypeStruct((NBLK, R, D), jnp.float32),
    )(X.reshape(NBLK, R, D), Ws, bc.reshape(-1, 1, D))


def _tc_rel_body(s_ref, p_ref, wr_ref, o_ref):
    acc = jnp.dot(s_ref[0], wr_ref[0], preferred_element_type=jnp.float32)
    o_ref[0] = jnp.maximum(acc + p_ref[0], 0.0)


def _tc_rel_call(S, P, Wr):
    out = pl.pallas_call(
        _tc_rel_body,
        grid=(NBLK,),
        in_specs=[
            pl.BlockSpec((1, R, D), lambda i: (jnp.maximum(i - SEG0_BLKS, 0), 0, 0)),
            pl.BlockSpec((1, R, D), lambda i: (i, 0, 0)),
            pl.BlockSpec((1, D, D), lambda i: (_seg(i), 0, 0)),
        ],
        out_specs=pl.BlockSpec((1, R, D), lambda i: (i, 0, 0)),
        out_shape=jax.ShapeDtypeStruct((NBLK, R, D), jnp.float32),
    )(S.reshape(-1, R, D), P, Wr)
    return out.reshape(N, D)


def _sc_geom(d):
    C = SC_CH[d]
    n = DEGS // C
    cpt = -(-n // SC_NW)
    cpt += cpt % 2
    return C, n, cpt


def _sc_body(feat, *rest):
    adjrs = rest[:MAX_DEG]
    s_out = rest[MAX_DEG]
    scratch = rest[MAX_DEG + 1:]
    ixs = scratch[:MAX_DEG]
    gbA, gbB, obA, obB, semI, semA, semB, semOA, semOB = scratch[MAX_DEG:]
    wid = lax.axis_index("s") * SC_NC + lax.axis_index("c")

    starts = {}
    for d in range(1, MAX_DEG + 1):
        C, n, cpt = _sc_geom(d)
        start = jnp.minimum(wid * cpt, n - cpt)
        starts[d] = start
        pltpu.async_copy(adjrs[d - 1].at[pl.ds(start, cpt)], ixs[d - 1], semI)
    for d in range(1, MAX_DEG + 1):
        C, n, cpt = _sc_geom(d)
        pltpu.make_async_copy(
            adjrs[d - 1].at[pl.ds(starts[d], cpt)], ixs[d - 1], semI).wait()

    for d in range(1, MAX_DEG + 1):
        C, n, cpt = _sc_geom(d)
        ix = ixs[d - 1]
        start = starts[d]
        npairs = cpt // 2

        def gather(t, gb, sem, d=d, C=C, ix=ix):
            pltpu.async_copy(feat.at[ix.at[t]], gb.at[pl.ds(0, C * d)], sem)

        def gwait(t, gb, sem, d=d, C=C, ix=ix):
            pltpu.make_async_copy(
                feat.at[ix.at[t]], gb.at[pl.ds(0, C * d)], sem).wait()

        def sumstore(t, gb, ob, semO, d=d, C=C, start=start):
            orow = (d - 1) * DEGS + (start + t) * C

            def body_j(j, _, d=d, gb=gb, ob=ob):
                rb = j * d
                for k in range(D // 16):
                    sl = pl.ds(k * 16, 16)
                    acc = gb[rb, sl]
                    for t2 in range(1, d):
                        acc = acc + gb[rb + t2, sl]
                    ob[j, sl] = acc
                return 0
            lax.fori_loop(0, C, body_j, 0)
            pltpu.async_copy(ob.at[pl.ds(0, C)], s_out.at[pl.ds(orow, C)], semO)

        def owait(t, ob, semO, d=d, C=C, start=start):
            orow = (d - 1) * DEGS + (start + t) * C
            pltpu.make_async_copy(
                ob.at[pl.ds(0, C)], s_out.at[pl.ds(orow, C)], semO).wait()

        gather(0, gbA, semA)

        def pair(p, _, npairs=npairs):
            t0 = 2 * p
            gather(t0 + 1, gbB, semB)
            gwait(t0, gbA, semA)

            @pl.when(p > 0)
            def _():
                owait(t0 - 2, obA, semOA)

            sumstore(t0, gbA, obA, semOA)

            @pl.when(p < npairs - 1)
            def _():
                gather(t0 + 2, gbA, semA)

            gwait(t0 + 1, gbB, semB)

            @pl.when(p > 0)
            def _():
                owait(t0 - 1, obB, semOB)

            sumstore(t0 + 1, gbB, obB, semOB)
            return 0

        lax.fori_loop(0, npairs, pair, 0)
        owait(cpt - 2, obA, semOA)
        owait(cpt - 1, obB, semOB)


def _sc_call(feat, adjs):
    adjrs = [a.reshape(DEGS // SC_CH[d + 1], SC_CH[d + 1] * (d + 1))
             for d, a in enumerate(adjs)]
    ix_types = []
    for d in range(1, MAX_DEG + 1):
        C, _, cpt = _sc_geom(d)
        ix_types.append(pltpu.VMEM((cpt, C * d), jnp.int32))
    fn = pl.kernel(
        _sc_body,
        out_type=jax.ShapeDtypeStruct((MAX_DEG * DEGS, D), jnp.float32),
        mesh=plsc.VectorSubcoreMesh(core_axis_name="c", subcore_axis_name="s"),
        scratch_types=ix_types + [
            pltpu.VMEM((128, D), jnp.float32),
            pltpu.VMEM((128, D), jnp.float32),
            pltpu.VMEM((60, D), jnp.float32),
            pltpu.VMEM((60, D), jnp.float32),
            pltpu.SemaphoreType.DMA,
            pltpu.SemaphoreType.DMA,
            pltpu.SemaphoreType.DMA,
            pltpu.SemaphoreType.DMA,
            pltpu.SemaphoreType.DMA,
        ],
        compiler_params=pltpu.CompilerParams(use_tc_tiling_on_sc=False),
    )
    return fn(feat, *adjrs)


def kernel(atom_features, deg_slice, adj_1, adj_2, adj_3, adj_4, adj_5,
           adj_6, adj_7, adj_8, adj_9, adj_10, W, b):
    adjs = [adj_1, adj_2, adj_3, adj_4, adj_5, adj_6, adj_7, adj_8, adj_9, adj_10]
    S = _sc_call(atom_features, adjs)

    Wr = jnp.concatenate([jnp.zeros((1, D, D), jnp.float32), W[0:20:2]], axis=0)
    Ws = jnp.concatenate([W[20:21], W[1:20:2]], axis=0)
    bc = jnp.concatenate([b[20:21], b[0:20:2] + b[1:20:2]], axis=0)
    P = _tc_self_call(atom_features, Ws, bc)
    return _tc_rel_call(S, P, Wr)
